# Initial kernel scaffold; baseline (speedup 1.0000x reference)
import jax
import jax.numpy as jnp
from jax.experimental import pallas as pl


def kernel(corr_index, rela_ht, rela_ct, nei_index, W_emb, b_emb, W_ih, b_ih, W_hh, b_hh):
    raise NotImplementedError("write your pallas kernel here")


# speedup vs baseline: 1.0113x; 1.0113x over previous
import jax
import jax.numpy as jnp
from jax.experimental import pallas as pl
from jax.experimental.pallas import tpu as pltpu

P = 512
EMB = 32
H = 64
N = P * P
R = 2048


def _lstm_block(corr_ref, ht_ref, ct_ref, mask_ref,
                wemb_ref, bemb_ref, wih_ref, whh_ref, bias_ref,
                hout_ref, cout_ref):
    corr = corr_ref[...]
    ht = ht_ref[...]
    ct = ct_ref[...]
    m = mask_ref[...]

    emb = jnp.maximum(
        corr[:, 0:1] * wemb_ref[0:1, :] + corr[:, 1:2] * wemb_ref[1:2, :]
        + bemb_ref[...], 0.0)

    gates = (jnp.dot(emb, wih_ref[...], preferred_element_type=jnp.float32)
             + jnp.dot(ht, whh_ref[...], preferred_element_type=jnp.float32)
             + bias_ref[...])

    i_g = jax.nn.sigmoid(gates[:, 0 * H:1 * H])
    f_g = jax.nn.sigmoid(gates[:, 1 * H:2 * H])
    g_g = jnp.tanh(gates[:, 2 * H:3 * H])
    o_g = jax.nn.sigmoid(gates[:, 3 * H:4 * H])

    c_new = f_g * ct + i_g * g_g
    h_new = o_g * jnp.tanh(c_new)

    hout_ref[...] = ht + m * (h_new - ht)
    cout_ref[...] = ct + m * (c_new - ct)


def kernel(corr_index, rela_ht, rela_ct, nei_index, W_emb, b_emb, W_ih, b_ih, W_hh, b_hh):
    corr = corr_index.reshape(N, 2)
    ht = rela_ht.reshape(N, H)
    ct = rela_ct.reshape(N, H)
    mask = (nei_index.reshape(N, 1) > 0).astype(jnp.float32)

    wemb = W_emb.T
    bemb = b_emb.reshape(1, EMB)
    wih = W_ih.T
    whh = W_hh.T
    bias = (b_ih + b_hh).reshape(1, 4 * H)

    grid = (N // R,)
    row_spec = lambda w: pl.BlockSpec((R, w), lambda i: (i, 0))
    full = lambda a: pl.BlockSpec(a.shape, lambda i: (0, 0))

    hout, cout = pl.pallas_call(
        _lstm_block,
        grid=grid,
        in_specs=[
            row_spec(2), row_spec(H), row_spec(H), row_spec(1),
            full(wemb), full(bemb), full(wih), full(whh), full(bias),
        ],
        out_specs=[row_spec(H), row_spec(H)],
        out_shape=[
            jax.ShapeDtypeStruct((N, H), jnp.float32),
            jax.ShapeDtypeStruct((N, H), jnp.float32),
        ],
        compiler_params=pltpu.CompilerParams(
            dimension_semantics=("arbitrary",),
        ),
    )(corr, ht, ct, mask, wemb, bemb, wih, whh, bias)

    return (hout.reshape(P, P, H), cout.reshape(P, P, H))
